# Initial kernel scaffold; baseline (speedup 1.0000x reference)
import jax
import jax.numpy as jnp
from jax.experimental import pallas as pl


def kernel(theta, cond, in_W, in_b, layers):
    raise NotImplementedError("write your pallas kernel here")


# speedup vs baseline: 2.9804x; 2.9804x over previous
import jax
import jax.numpy as jnp
from jax.experimental import pallas as pl

_J = 53
_FB = 40
_ROWS = _FB * _J


def _encoder_body(layer_desc, *refs):
    theta_ref, cond_ref, in_w_ref, in_b_ref = refs[:4]
    out_ref = refs[-1]
    lrefs = refs[4:-1]

    x = theta_ref[...]
    h = jnp.dot(x, in_w_ref[...],
                preferred_element_type=jnp.float32) + in_b_ref[...]
    cond = cond_ref[...].reshape(1, -1)

    i = 0
    for (_di, _do, has_res) in layer_desc:
        pose = lrefs[i][...]; i += 1
        ln_g = lrefs[i][...]; i += 1
        ln_b = lrefs[i][...]; i += 1
        if has_res:
            res_w = lrefs[i][...]; i += 1
        g_w = lrefs[i][...]; i += 1
        g_b = lrefs[i][...]; i += 1
        b_w = lrefs[i][...]; i += 1
        b_b = lrefs[i][...]; i += 1

        mu = jnp.mean(pose, axis=-1, keepdims=True)
        var = jnp.mean(jnp.square(pose - mu), axis=-1, keepdims=True)
        normed = (pose - mu) * jax.lax.rsqrt(var + 1e-5) * ln_g + ln_b
        c = 0.5 * normed * (1.0 + jax.lax.erf(normed * (2.0 ** -0.5)))

        g = jax.nn.sigmoid(
            jnp.dot(cond, g_w, preferred_element_type=jnp.float32) + g_b)
        bta = jnp.tanh(
            jnp.dot(cond, b_w, preferred_element_type=jnp.float32) + b_b)

        res = (jnp.dot(h, res_w, preferred_element_type=jnp.float32)
               if has_res else h)
        c_tiled = jnp.concatenate([c] * _FB, axis=0)
        h = (c_tiled + res) * g + bta

    out_ref[...] = h


def kernel(theta, cond, in_W, in_b, layers):
    B, F, J, nf = theta.shape
    cond_dim = cond.shape[-1]
    tot = B * F * J
    nblk = tot // _ROWS
    blk_per_batch = F // _FB

    theta2 = theta.reshape(tot, nf)
    cond3 = cond.reshape(B, 1, cond_dim)

    layer_desc = []
    operands = [theta2, cond3, in_W, in_b.reshape(1, -1)]
    specs = [
        pl.BlockSpec((_ROWS, nf), lambda i: (i, 0)),
        pl.BlockSpec((1, 1, cond_dim), lambda i: (i // blk_per_batch, 0, 0)),
        pl.BlockSpec(in_W.shape, lambda i: (0, 0)),
        pl.BlockSpec((1, in_b.shape[0]), lambda i: (0, 0)),
    ]

    def add_full(arr):
        a2 = arr.reshape(1, -1) if arr.ndim == 1 else arr
        operands.append(a2)
        specs.append(pl.BlockSpec(a2.shape, lambda i: (0,) * a2.ndim))

    for p in layers:
        do = p["pose"].shape[-1]
        has_res = p["res_W"] is not None
        di = p["res_W"].shape[0] if has_res else do
        layer_desc.append((di, do, has_res))
        add_full(p["pose"])
        add_full(p["ln_g"])
        add_full(p["ln_b"])
        if has_res:
            add_full(p["res_W"])
        add_full(p["g_W"])
        add_full(p["g_b"])
        add_full(p["b_W"])
        add_full(p["b_b"])

    d_out = layers[-1]["pose"].shape[-1]
    out = pl.pallas_call(
        lambda *refs: _encoder_body(tuple(layer_desc), *refs),
        grid=(nblk,),
        in_specs=specs,
        out_specs=pl.BlockSpec((_ROWS, d_out), lambda i: (i, 0)),
        out_shape=jax.ShapeDtypeStruct((tot, d_out), jnp.float32),
    )(*operands)
    return out.reshape(B, F, J, d_out)


# speedup vs baseline: 4.1317x; 1.3863x over previous
import jax
import jax.numpy as jnp
from jax.experimental import pallas as pl

_J = 53
_JP = 56
_FB = 40
_ROWS = _FB * _JP


def _encoder_body(layer_desc, *refs):
    theta_ref, cond_ref, in_w_ref, in_b_ref = refs[:4]
    out_ref = refs[-1]
    lrefs = refs[4:-1]

    x = theta_ref[...]
    h = jnp.dot(x, in_w_ref[...],
                preferred_element_type=jnp.float32) + in_b_ref[...]
    cond = cond_ref[...].reshape(1, -1)

    i = 0
    for (_di, _do, has_res) in layer_desc:
        pose = lrefs[i][...]; i += 1
        ln_g = lrefs[i][...]; i += 1
        ln_b = lrefs[i][...]; i += 1
        if has_res:
            res_w = lrefs[i][...]; i += 1
        g_w = lrefs[i][...]; i += 1
        g_b = lrefs[i][...]; i += 1
        b_w = lrefs[i][...]; i += 1
        b_b = lrefs[i][...]; i += 1

        mu = jnp.mean(pose, axis=-1, keepdims=True)
        var = jnp.mean(jnp.square(pose - mu), axis=-1, keepdims=True)
        normed = (pose - mu) * jax.lax.rsqrt(var + 1e-5) * ln_g + ln_b
        c = 0.5 * normed * (1.0 + jax.lax.erf(normed * (2.0 ** -0.5)))

        g = jax.nn.sigmoid(
            jnp.dot(cond, g_w, preferred_element_type=jnp.float32) + g_b)
        bta = jnp.tanh(
            jnp.dot(cond, b_w, preferred_element_type=jnp.float32) + b_b)

        res = (jnp.dot(h, res_w, preferred_element_type=jnp.float32)
               if has_res else h)
        cp = jnp.concatenate(
            [c, jnp.zeros((_JP - _J, c.shape[-1]), c.dtype)], axis=0)
        c_tiled = jnp.concatenate([cp] * _FB, axis=0)
        h = (c_tiled + res) * g + bta

    h3 = h.reshape(_FB, _JP, h.shape[-1])
    out_ref[...] = h3[:, :_J, :]


def kernel(theta, cond, in_W, in_b, layers):
    B, F, J, nf = theta.shape
    cond_dim = cond.shape[-1]
    nblk = (B * F) // _FB
    blk_per_batch = F // _FB

    theta_p = jnp.pad(theta, ((0, 0), (0, 0), (0, _JP - J), (0, 0)))
    theta2 = theta_p.reshape(B * F * _JP, nf)
    cond3 = cond.reshape(B, 1, cond_dim)

    layer_desc = []
    operands = [theta2, cond3, in_W, in_b.reshape(1, -1)]
    specs = [
        pl.BlockSpec((_ROWS, nf), lambda i: (i, 0)),
        pl.BlockSpec((1, 1, cond_dim), lambda i: (i // blk_per_batch, 0, 0)),
        pl.BlockSpec(in_W.shape, lambda i: (0, 0)),
        pl.BlockSpec((1, in_b.shape[0]), lambda i: (0, 0)),
    ]

    def add_full(arr):
        a2 = arr.reshape(1, -1) if arr.ndim == 1 else arr
        operands.append(a2)
        specs.append(pl.BlockSpec(a2.shape, lambda i: (0,) * a2.ndim))

    for p in layers:
        do = p["pose"].shape[-1]
        has_res = p["res_W"] is not None
        di = p["res_W"].shape[0] if has_res else do
        layer_desc.append((di, do, has_res))
        add_full(p["pose"])
        add_full(p["ln_g"])
        add_full(p["ln_b"])
        if has_res:
            add_full(p["res_W"])
        add_full(p["g_W"])
        add_full(p["g_b"])
        add_full(p["b_W"])
        add_full(p["b_b"])

    d_out = layers[-1]["pose"].shape[-1]
    out = pl.pallas_call(
        lambda *refs: _encoder_body(tuple(layer_desc), *refs),
        grid=(nblk,),
        in_specs=specs,
        out_specs=pl.BlockSpec((_FB, J, d_out), lambda i: (i, 0, 0)),
        out_shape=jax.ShapeDtypeStruct((B * F, J, d_out), jnp.float32),
    )(*operands)
    return out.reshape(B, F, J, d_out)


# speedup vs baseline: 4.5455x; 1.1001x over previous
import jax
import jax.numpy as jnp
from jax.experimental import pallas as pl

_J = 53
_JP = 56
_FB = 40
_ROWS = _FB * _JP


def _encoder_body(layer_desc, *refs):
    theta_ref, cond_ref, in_w_ref, in_b_ref = refs[:4]
    out_ref = refs[-1]
    lrefs = refs[4:-1]

    x3 = theta_ref[...]
    x3p = jnp.pad(x3, ((0, 0), (0, _JP - _J), (0, 0)))
    x = x3p.reshape(_ROWS, x3.shape[-1])
    h = jnp.dot(x, in_w_ref[...],
                preferred_element_type=jnp.float32) + in_b_ref[...]
    cond = cond_ref[...].reshape(1, -1)

    i = 0
    for (_di, _do, has_res) in layer_desc:
        pose = lrefs[i][...]; i += 1
        ln_g = lrefs[i][...]; i += 1
        ln_b = lrefs[i][...]; i += 1
        if has_res:
            res_w = lrefs[i][...]; i += 1
        g_w = lrefs[i][...]; i += 1
        g_b = lrefs[i][...]; i += 1
        b_w = lrefs[i][...]; i += 1
        b_b = lrefs[i][...]; i += 1

        mu = jnp.mean(pose, axis=-1, keepdims=True)
        var = jnp.mean(jnp.square(pose - mu), axis=-1, keepdims=True)
        normed = (pose - mu) * jax.lax.rsqrt(var + 1e-5) * ln_g + ln_b
        c = 0.5 * normed * (1.0 + jax.lax.erf(normed * (2.0 ** -0.5)))

        g = jax.nn.sigmoid(
            jnp.dot(cond, g_w, preferred_element_type=jnp.float32) + g_b)
        bta = jnp.tanh(
            jnp.dot(cond, b_w, preferred_element_type=jnp.float32) + b_b)

        res = (jnp.dot(h, res_w, preferred_element_type=jnp.float32)
               if has_res else h)
        cp = jnp.concatenate(
            [c, jnp.zeros((_JP - _J, c.shape[-1]), c.dtype)], axis=0)
        c_tiled = jnp.concatenate([cp] * _FB, axis=0)
        h = (c_tiled + res) * g + bta

    h3 = h.reshape(_FB, _JP, h.shape[-1])
    out_ref[...] = h3[:, :_J, :]


def kernel(theta, cond, in_W, in_b, layers):
    B, F, J, nf = theta.shape
    cond_dim = cond.shape[-1]
    nblk = (B * F) // _FB
    blk_per_batch = F // _FB

    theta2 = theta.reshape(B * F, J, nf)
    cond3 = cond.reshape(B, 1, cond_dim)

    layer_desc = []
    operands = [theta2, cond3, in_W, in_b.reshape(1, -1)]
    specs = [
        pl.BlockSpec((_FB, J, nf), lambda i: (i, 0, 0)),
        pl.BlockSpec((1, 1, cond_dim), lambda i: (i // blk_per_batch, 0, 0)),
        pl.BlockSpec(in_W.shape, lambda i: (0, 0)),
        pl.BlockSpec((1, in_b.shape[0]), lambda i: (0, 0)),
    ]

    def add_full(arr):
        a2 = arr.reshape(1, -1) if arr.ndim == 1 else arr
        operands.append(a2)
        specs.append(pl.BlockSpec(a2.shape, lambda i: (0,) * a2.ndim))

    for p in layers:
        do = p["pose"].shape[-1]
        has_res = p["res_W"] is not None
        di = p["res_W"].shape[0] if has_res else do
        layer_desc.append((di, do, has_res))
        add_full(p["pose"])
        add_full(p["ln_g"])
        add_full(p["ln_b"])
        if has_res:
            add_full(p["res_W"])
        add_full(p["g_W"])
        add_full(p["g_b"])
        add_full(p["b_W"])
        add_full(p["b_b"])

    d_out = layers[-1]["pose"].shape[-1]
    out = pl.pallas_call(
        lambda *refs: _encoder_body(tuple(layer_desc), *refs),
        grid=(nblk,),
        in_specs=specs,
        out_specs=pl.BlockSpec((_FB, J, d_out), lambda i: (i, 0, 0)),
        out_shape=jax.ShapeDtypeStruct((B * F, J, d_out), jnp.float32),
    )(*operands)
    return out.reshape(B, F, J, d_out)


# speedup vs baseline: 7.3971x; 1.6274x over previous
import jax
import jax.numpy as jnp
from jax.experimental import pallas as pl

_JB = 28


def _encoder_body(layer_desc, n_joints, n_frames, *refs):
    theta_ref, cond_ref, in_w_ref, in_b_ref = refs[:4]
    out_ref = refs[-1]
    lrefs = refs[4:-1]
    rows = _JB * n_frames

    x3 = theta_ref[...]
    x = x3.reshape(rows, x3.shape[-1])
    h = jnp.dot(x, in_w_ref[...],
                preferred_element_type=jnp.float32) + in_b_ref[...]
    cond = cond_ref[...].reshape(1, -1)

    j0 = pl.program_id(1) * _JB
    row_j = j0 + jax.lax.broadcasted_iota(jnp.int32, (rows, n_joints), 0) \
        // n_frames
    col_j = jax.lax.broadcasted_iota(jnp.int32, (rows, n_joints), 1)
    sel = jnp.where(row_j == col_j, 1.0, 0.0).astype(jnp.float32)

    i = 0
    for (_di, _do, has_res) in layer_desc:
        pose = lrefs[i][...]; i += 1
        ln_g = lrefs[i][...]; i += 1
        ln_b = lrefs[i][...]; i += 1
        if has_res:
            res_w = lrefs[i][...]; i += 1
        g_w = lrefs[i][...]; i += 1
        g_b = lrefs[i][...]; i += 1
        b_w = lrefs[i][...]; i += 1
        b_b = lrefs[i][...]; i += 1

        mu = jnp.mean(pose, axis=-1, keepdims=True)
        var = jnp.mean(jnp.square(pose - mu), axis=-1, keepdims=True)
        normed = (pose - mu) * jax.lax.rsqrt(var + 1e-5) * ln_g + ln_b
        c = 0.5 * normed * (1.0 + jax.lax.erf(normed * (2.0 ** -0.5)))

        g = jax.nn.sigmoid(
            jnp.dot(cond, g_w, preferred_element_type=jnp.float32) + g_b)
        bta = jnp.tanh(
            jnp.dot(cond, b_w, preferred_element_type=jnp.float32) + b_b)

        res = (jnp.dot(h, res_w, preferred_element_type=jnp.float32)
               if has_res else h)
        c_rows = jnp.dot(sel, c, preferred_element_type=jnp.float32)
        h = (c_rows + res) * g + bta

    out_ref[...] = h.reshape(1, _JB, n_frames, h.shape[-1])


def kernel(theta, cond, in_W, in_b, layers):
    B, F, J, nf = theta.shape
    cond_dim = cond.shape[-1]
    nj = -(-J // _JB)

    theta_t = jnp.transpose(theta, (0, 2, 1, 3))
    cond3 = cond.reshape(B, 1, cond_dim)

    layer_desc = []
    operands = [theta_t, cond3, in_W, in_b.reshape(1, -1)]
    specs = [
        pl.BlockSpec((1, _JB, F, nf), lambda b, jc: (b, jc, 0, 0)),
        pl.BlockSpec((1, 1, cond_dim), lambda b, jc: (b, 0, 0)),
        pl.BlockSpec(in_W.shape, lambda b, jc: (0, 0)),
        pl.BlockSpec((1, in_b.shape[0]), lambda b, jc: (0, 0)),
    ]

    def add_full(arr):
        a2 = arr.reshape(1, -1) if arr.ndim == 1 else arr
        operands.append(a2)
        specs.append(pl.BlockSpec(a2.shape, lambda b, jc: (0,) * a2.ndim))

    for p in layers:
        do = p["pose"].shape[-1]
        has_res = p["res_W"] is not None
        di = p["res_W"].shape[0] if has_res else do
        layer_desc.append((di, do, has_res))
        add_full(p["pose"])
        add_full(p["ln_g"])
        add_full(p["ln_b"])
        if has_res:
            add_full(p["res_W"])
        add_full(p["g_W"])
        add_full(p["g_b"])
        add_full(p["b_W"])
        add_full(p["b_b"])

    d_out = layers[-1]["pose"].shape[-1]
    out = pl.pallas_call(
        lambda *refs: _encoder_body(tuple(layer_desc), J, F, *refs),
        grid=(B, nj),
        in_specs=specs,
        out_specs=pl.BlockSpec((1, _JB, F, d_out), lambda b, jc: (b, jc, 0, 0)),
        out_shape=jax.ShapeDtypeStruct((B, J, F, d_out), jnp.float32),
    )(*operands)
    return jnp.transpose(out, (0, 2, 1, 3))


# speedup vs baseline: 8.9314x; 1.2074x over previous
import jax
import jax.numpy as jnp
from jax.experimental import pallas as pl
from jax.experimental.pallas import tpu as pltpu

_JB = 28


def _encoder_body(layer_desc, n_joints, n_frames, *refs):
    nl = len(layer_desc)
    theta_ref, cond_ref, in_w_ref, in_b_ref = refs[:4]
    out_ref = refs[-1 - nl]
    crefs = refs[-nl:]
    lrefs = refs[4:-1 - nl]
    rows = _JB * n_frames

    @pl.when(pl.program_id(1) == 0)
    def _compute_constants():
        j0 = pl.program_id(0) * _JB
        row_j = j0 + jax.lax.broadcasted_iota(
            jnp.int32, (rows, n_joints), 0) // n_frames
        col_j = jax.lax.broadcasted_iota(jnp.int32, (rows, n_joints), 1)
        sel = jnp.where(row_j == col_j, 1.0, 0.0).astype(jnp.float32)
        i = 0
        for li, (_di, _do, has_res) in enumerate(layer_desc):
            pose = lrefs[i][...]; i += 1
            ln_g = lrefs[i][...]; i += 1
            ln_b = lrefs[i][...]; i += 1
            if has_res:
                i += 1
            i += 4
            mu = jnp.mean(pose, axis=-1, keepdims=True)
            var = jnp.mean(jnp.square(pose - mu), axis=-1, keepdims=True)
            normed = (pose - mu) * jax.lax.rsqrt(var + 1e-5) * ln_g + ln_b
            c = 0.5 * normed * (1.0 + jax.lax.erf(normed * (2.0 ** -0.5)))
            crefs[li][...] = jnp.dot(sel, c,
                                     preferred_element_type=jnp.float32)

    x3 = theta_ref[...]
    x = x3.reshape(rows, x3.shape[-1])
    h = jnp.dot(x, in_w_ref[...],
                preferred_element_type=jnp.float32) + in_b_ref[...]
    cond = cond_ref[...].reshape(1, -1)

    i = 0
    for li, (_di, _do, has_res) in enumerate(layer_desc):
        i += 3
        if has_res:
            res_w = lrefs[i][...]; i += 1
        g_w = lrefs[i][...]; i += 1
        g_b = lrefs[i][...]; i += 1
        b_w = lrefs[i][...]; i += 1
        b_b = lrefs[i][...]; i += 1

        g = jax.nn.sigmoid(
            jnp.dot(cond, g_w, preferred_element_type=jnp.float32) + g_b)
        bta = jnp.tanh(
            jnp.dot(cond, b_w, preferred_element_type=jnp.float32) + b_b)

        res = (jnp.dot(h, res_w, preferred_element_type=jnp.float32)
               if has_res else h)
        h = (crefs[li][...] + res) * g + bta

    out_ref[...] = h.reshape(1, _JB, n_frames, h.shape[-1])


def kernel(theta, cond, in_W, in_b, layers):
    B, F, J, nf = theta.shape
    cond_dim = cond.shape[-1]
    nj = -(-J // _JB)

    theta_t = jnp.transpose(theta, (0, 2, 1, 3))
    cond3 = cond.reshape(B, 1, cond_dim)

    layer_desc = []
    operands = [theta_t, cond3, in_W, in_b.reshape(1, -1)]
    specs = [
        pl.BlockSpec((1, _JB, F, nf), lambda jc, b: (b, jc, 0, 0)),
        pl.BlockSpec((1, 1, cond_dim), lambda jc, b: (b, 0, 0)),
        pl.BlockSpec(in_W.shape, lambda jc, b: (0, 0)),
        pl.BlockSpec((1, in_b.shape[0]), lambda jc, b: (0, 0)),
    ]

    def add_full(arr):
        a2 = arr.reshape(1, -1) if arr.ndim == 1 else arr
        operands.append(a2)
        specs.append(pl.BlockSpec(a2.shape, lambda jc, b: (0,) * a2.ndim))

    for p in layers:
        do = p["pose"].shape[-1]
        has_res = p["res_W"] is not None
        di = p["res_W"].shape[0] if has_res else do
        layer_desc.append((di, do, has_res))
        add_full(p["pose"])
        add_full(p["ln_g"])
        add_full(p["ln_b"])
        if has_res:
            add_full(p["res_W"])
        add_full(p["g_W"])
        add_full(p["g_b"])
        add_full(p["b_W"])
        add_full(p["b_b"])

    d_out = layers[-1]["pose"].shape[-1]
    rows = _JB * F
    out = pl.pallas_call(
        lambda *refs: _encoder_body(tuple(layer_desc), J, F, *refs),
        grid=(nj, B),
        in_specs=specs,
        out_specs=pl.BlockSpec((1, _JB, F, d_out), lambda jc, b: (b, jc, 0, 0)),
        out_shape=jax.ShapeDtypeStruct((B, J, F, d_out), jnp.float32),
        scratch_shapes=[pltpu.VMEM((rows, do), jnp.float32)
                        for (_di, do, _hr) in layer_desc],
    )(*operands)
    return jnp.transpose(out, (0, 2, 1, 3))


# speedup vs baseline: 9.0519x; 1.0135x over previous
import jax
import jax.numpy as jnp
from jax.experimental import pallas as pl
from jax.experimental.pallas import tpu as pltpu

_JB = 28


def _encoder_body(layer_desc, n_joints, n_frames, *refs):
    nl = len(layer_desc)
    theta_ref, cond_ref, in_w_ref, in_b_ref = refs[:4]
    out_ref = refs[-1 - nl]
    crefs = refs[-nl:]
    lrefs = refs[4:-1 - nl]
    rows = _JB * n_frames

    @pl.when(pl.program_id(1) == 0)
    def _compute_constants():
        j0 = pl.program_id(0) * _JB
        row_j = j0 + jax.lax.broadcasted_iota(
            jnp.int32, (rows, n_joints), 0) // n_frames
        col_j = jax.lax.broadcasted_iota(jnp.int32, (rows, n_joints), 1)
        sel = jnp.where(row_j == col_j, 1.0, 0.0).astype(jnp.float32)
        i = 0
        for li, (_di, _do, has_res) in enumerate(layer_desc):
            pose = lrefs[i][...]; i += 1
            ln_g = lrefs[i][...]; i += 1
            ln_b = lrefs[i][...]; i += 1
            if has_res:
                i += 1
            i += 4
            mu = jnp.mean(pose, axis=-1, keepdims=True)
            var = jnp.mean(jnp.square(pose - mu), axis=-1, keepdims=True)
            normed = (pose - mu) * jax.lax.rsqrt(var + 1e-5) * ln_g + ln_b
            c = 0.5 * normed * (1.0 + jax.lax.erf(normed * (2.0 ** -0.5)))
            crefs[li][...] = jnp.dot(sel, c,
                                     preferred_element_type=jnp.float32)

    x3 = theta_ref[...]
    x = x3.reshape(rows, x3.shape[-1])
    h = jnp.dot(x, in_w_ref[...],
                preferred_element_type=jnp.float32) + in_b_ref[...]
    cond = cond_ref[...].reshape(1, -1)

    i = 0
    for li, (_di, _do, has_res) in enumerate(layer_desc):
        i += 3
        if has_res:
            res_w = lrefs[i][...]; i += 1
        g_w = lrefs[i][...]; i += 1
        g_b = lrefs[i][...]; i += 1
        b_w = lrefs[i][...]; i += 1
        b_b = lrefs[i][...]; i += 1

        g = jax.nn.sigmoid(
            jnp.dot(cond, g_w, preferred_element_type=jnp.float32) + g_b)
        bta = jnp.tanh(
            jnp.dot(cond, b_w, preferred_element_type=jnp.float32) + b_b)

        res = (jnp.dot(h.astype(res_w.dtype), res_w,
                       preferred_element_type=jnp.float32)
               if has_res else h)
        h = (crefs[li][...] + res) * g + bta

    out_ref[...] = h.reshape(1, _JB, n_frames, h.shape[-1])


def kernel(theta, cond, in_W, in_b, layers):
    B, F, J, nf = theta.shape
    cond_dim = cond.shape[-1]
    nj = -(-J // _JB)

    theta_t = jnp.transpose(theta, (0, 2, 1, 3))
    cond3 = cond.reshape(B, 1, cond_dim)

    layer_desc = []
    operands = [theta_t, cond3, in_W, in_b.reshape(1, -1)]
    specs = [
        pl.BlockSpec((1, _JB, F, nf), lambda jc, b: (b, jc, 0, 0)),
        pl.BlockSpec((1, 1, cond_dim), lambda jc, b: (b, 0, 0)),
        pl.BlockSpec(in_W.shape, lambda jc, b: (0, 0)),
        pl.BlockSpec((1, in_b.shape[0]), lambda jc, b: (0, 0)),
    ]

    def add_full(arr):
        a2 = arr.reshape(1, -1) if arr.ndim == 1 else arr
        operands.append(a2)
        specs.append(pl.BlockSpec(a2.shape, lambda jc, b: (0,) * a2.ndim))

    for p in layers:
        do = p["pose"].shape[-1]
        has_res = p["res_W"] is not None
        di = p["res_W"].shape[0] if has_res else do
        layer_desc.append((di, do, has_res))
        add_full(p["pose"])
        add_full(p["ln_g"])
        add_full(p["ln_b"])
        if has_res:
            add_full(p["res_W"].astype(jnp.bfloat16))
        add_full(p["g_W"])
        add_full(p["g_b"])
        add_full(p["b_W"])
        add_full(p["b_b"])

    d_out = layers[-1]["pose"].shape[-1]
    rows = _JB * F
    out = pl.pallas_call(
        lambda *refs: _encoder_body(tuple(layer_desc), J, F, *refs),
        grid=(nj, B),
        in_specs=specs,
        out_specs=pl.BlockSpec((1, _JB, F, d_out), lambda jc, b: (b, jc, 0, 0)),
        out_shape=jax.ShapeDtypeStruct((B, J, F, d_out), jnp.float32),
        scratch_shapes=[pltpu.VMEM((rows, do), jnp.float32)
                        for (_di, do, _hr) in layer_desc],
    )(*operands)
    return jnp.transpose(out, (0, 2, 1, 3))


# speedup vs baseline: 10.6631x; 1.1780x over previous
import jax
import jax.numpy as jnp
from jax.experimental import pallas as pl
from jax.experimental.pallas import tpu as pltpu

_JB = 18


def _encoder_body(layer_desc, n_joints, n_frames, n_batch, *refs):
    nl = len(layer_desc)
    theta_ref, cond_ref, in_w_ref, in_b_ref = refs[:4]
    out_ref = refs[-1 - 3 * nl]
    crefs = refs[-3 * nl:-2 * nl]
    grefs = refs[-2 * nl:-nl]
    brefs = refs[-nl:]
    lrefs = refs[4:-1 - 3 * nl]
    rows = _JB * n_frames
    jc = pl.program_id(0)
    b = pl.program_id(1)

    @pl.when(jnp.logical_and(jc == 0, b == 0))
    def _compute_gates():
        cond = cond_ref[...]
        i = 0
        for li, (_di, _do, has_res) in enumerate(layer_desc):
            i += 3
            if has_res:
                i += 1
            g_w = lrefs[i][...]; i += 1
            g_b = lrefs[i][...]; i += 1
            b_w = lrefs[i][...]; i += 1
            b_b = lrefs[i][...]; i += 1
            grefs[li][...] = jax.nn.sigmoid(
                jnp.dot(cond, g_w, preferred_element_type=jnp.float32) + g_b)
            brefs[li][...] = jnp.tanh(
                jnp.dot(cond, b_w, preferred_element_type=jnp.float32) + b_b)

    @pl.when(b == 0)
    def _compute_constants():
        j0 = jc * _JB
        row_j = j0 + jax.lax.broadcasted_iota(
            jnp.int32, (rows, n_joints), 0) // n_frames
        col_j = jax.lax.broadcasted_iota(jnp.int32, (rows, n_joints), 1)
        sel = jnp.where(row_j == col_j, 1.0, 0.0).astype(jnp.float32)
        i = 0
        for li, (_di, _do, has_res) in enumerate(layer_desc):
            pose = lrefs[i][...]; i += 1
            ln_g = lrefs[i][...]; i += 1
            ln_b = lrefs[i][...]; i += 1
            if has_res:
                i += 1
            i += 4
            mu = jnp.mean(pose, axis=-1, keepdims=True)
            var = jnp.mean(jnp.square(pose - mu), axis=-1, keepdims=True)
            normed = (pose - mu) * jax.lax.rsqrt(var + 1e-5) * ln_g + ln_b
            c = 0.5 * normed * (1.0 + jax.lax.erf(normed * (2.0 ** -0.5)))
            crefs[li][...] = jnp.dot(sel, c,
                                     preferred_element_type=jnp.float32)

    x3 = theta_ref[...]
    x = x3.reshape(rows, x3.shape[-1])
    h = jnp.dot(x, in_w_ref[...],
                preferred_element_type=jnp.float32) + in_b_ref[...]

    i = 0
    for li, (_di, _do, has_res) in enumerate(layer_desc):
        i += 3
        if has_res:
            res_w = lrefs[i][...]; i += 1
        i += 4
        g = grefs[li][pl.ds(b, 1), :]
        bta = brefs[li][pl.ds(b, 1), :]
        res = (jnp.dot(h.astype(res_w.dtype), res_w,
                       preferred_element_type=jnp.float32)
               if has_res else h)
        h = (crefs[li][...] + res) * g + bta

    out_ref[...] = h.reshape(1, _JB, n_frames, h.shape[-1])


def kernel(theta, cond, in_W, in_b, layers):
    B, F, J, nf = theta.shape
    cond_dim = cond.shape[-1]
    nj = -(-J // _JB)

    theta_t = jnp.transpose(theta, (0, 2, 1, 3))

    layer_desc = []
    operands = [theta_t, cond, in_W, in_b.reshape(1, -1)]
    specs = [
        pl.BlockSpec((1, _JB, F, nf), lambda jc, b: (b, jc, 0, 0)),
        pl.BlockSpec((B, cond_dim), lambda jc, b: (0, 0)),
        pl.BlockSpec(in_W.shape, lambda jc, b: (0, 0)),
        pl.BlockSpec((1, in_b.shape[0]), lambda jc, b: (0, 0)),
    ]

    def add_full(arr):
        a2 = arr.reshape(1, -1) if arr.ndim == 1 else arr
        operands.append(a2)
        specs.append(pl.BlockSpec(a2.shape, lambda jc, b: (0,) * a2.ndim))

    for p in layers:
        do = p["pose"].shape[-1]
        has_res = p["res_W"] is not None
        di = p["res_W"].shape[0] if has_res else do
        layer_desc.append((di, do, has_res))
        add_full(p["pose"])
        add_full(p["ln_g"])
        add_full(p["ln_b"])
        if has_res:
            add_full(p["res_W"].astype(jnp.bfloat16))
        add_full(p["g_W"])
        add_full(p["g_b"])
        add_full(p["b_W"])
        add_full(p["b_b"])

    d_out = layers[-1]["pose"].shape[-1]
    rows = _JB * F
    scratch = [pltpu.VMEM((rows, do), jnp.float32)
               for (_di, do, _hr) in layer_desc]
    scratch += [pltpu.VMEM((B, do), jnp.float32)
                for (_di, do, _hr) in layer_desc]
    scratch += [pltpu.VMEM((B, do), jnp.float32)
                for (_di, do, _hr) in layer_desc]
    out = pl.pallas_call(
        lambda *refs: _encoder_body(tuple(layer_desc), J, F, B, *refs),
        grid=(nj, B),
        in_specs=specs,
        out_specs=pl.BlockSpec((1, _JB, F, d_out), lambda jc, b: (b, jc, 0, 0)),
        out_shape=jax.ShapeDtypeStruct((B, J, F, d_out), jnp.float32),
        scratch_shapes=scratch,
    )(*operands)
    return jnp.transpose(out, (0, 2, 1, 3))


# speedup vs baseline: 11.2422x; 1.0543x over previous
import jax
import jax.numpy as jnp
from jax.experimental import pallas as pl
from jax.experimental.pallas import tpu as pltpu

_JB = 27


def _encoder_body(layer_desc, n_joints, n_frames, n_batch, *refs):
    nl = len(layer_desc)
    theta_ref, cond_ref, in_w_ref, in_b_ref = refs[:4]
    out_ref = refs[-1 - 3 * nl]
    crefs = refs[-3 * nl:-2 * nl]
    grefs = refs[-2 * nl:-nl]
    brefs = refs[-nl:]
    lrefs = refs[4:-1 - 3 * nl]
    rows = _JB * n_frames
    jc = pl.program_id(0)
    b = pl.program_id(1)

    @pl.when(jnp.logical_and(jc == 0, b == 0))
    def _compute_gates():
        cond = cond_ref[...]
        i = 0
        for li, (_di, _do, has_res) in enumerate(layer_desc):
            i += 3
            if has_res:
                i += 1
            g_w = lrefs[i][...]; i += 1
            g_b = lrefs[i][...]; i += 1
            b_w = lrefs[i][...]; i += 1
            b_b = lrefs[i][...]; i += 1
            grefs[li][...] = jax.nn.sigmoid(
                jnp.dot(cond, g_w, preferred_element_type=jnp.float32) + g_b)
            brefs[li][...] = jnp.tanh(
                jnp.dot(cond, b_w, preferred_element_type=jnp.float32) + b_b)

    @pl.when(b == 0)
    def _compute_constants():
        j0 = jc * _JB
        row_j = j0 + jax.lax.broadcasted_iota(
            jnp.int32, (rows, n_joints), 0) // n_frames
        col_j = jax.lax.broadcasted_iota(jnp.int32, (rows, n_joints), 1)
        sel = jnp.where(row_j == col_j, 1.0, 0.0).astype(jnp.float32)
        i = 0
        for li, (_di, _do, has_res) in enumerate(layer_desc):
            pose = lrefs[i][...]; i += 1
            ln_g = lrefs[i][...]; i += 1
            ln_b = lrefs[i][...]; i += 1
            if has_res:
                i += 1
            i += 4
            mu = jnp.mean(pose, axis=-1, keepdims=True)
            var = jnp.mean(jnp.square(pose - mu), axis=-1, keepdims=True)
            normed = (pose - mu) * jax.lax.rsqrt(var + 1e-5) * ln_g + ln_b
            c = 0.5 * normed * (1.0 + jax.lax.erf(normed * (2.0 ** -0.5)))
            crefs[li][...] = jnp.dot(sel, c,
                                     preferred_element_type=jnp.float32)

    x3 = theta_ref[...]
    x = x3.reshape(rows, x3.shape[-1])
    h = jnp.dot(x, in_w_ref[...],
                preferred_element_type=jnp.float32) + in_b_ref[...]

    i = 0
    for li, (_di, _do, has_res) in enumerate(layer_desc):
        i += 3
        if has_res:
            res_w = lrefs[i][...]; i += 1
        i += 4
        g = grefs[li][pl.ds(b, 1), :]
        bta = brefs[li][pl.ds(b, 1), :]
        res = (jnp.dot(h.astype(res_w.dtype), res_w,
                       preferred_element_type=jnp.float32)
               if has_res else h)
        h = (crefs[li][...] + res) * g + bta

    out_ref[...] = h.reshape(1, _JB, n_frames, h.shape[-1])


def kernel(theta, cond, in_W, in_b, layers):
    B, F, J, nf = theta.shape
    cond_dim = cond.shape[-1]
    nj = -(-J // _JB)

    theta_t = jnp.transpose(theta, (0, 2, 1, 3))

    layer_desc = []
    operands = [theta_t, cond, in_W, in_b.reshape(1, -1)]
    specs = [
        pl.BlockSpec((1, _JB, F, nf), lambda jc, b: (b, jc, 0, 0)),
        pl.BlockSpec((B, cond_dim), lambda jc, b: (0, 0)),
        pl.BlockSpec(in_W.shape, lambda jc, b: (0, 0)),
        pl.BlockSpec((1, in_b.shape[0]), lambda jc, b: (0, 0)),
    ]

    def add_full(arr):
        a2 = arr.reshape(1, -1) if arr.ndim == 1 else arr
        operands.append(a2)
        specs.append(pl.BlockSpec(a2.shape, lambda jc, b: (0,) * a2.ndim))

    for p in layers:
        do = p["pose"].shape[-1]
        has_res = p["res_W"] is not None
        di = p["res_W"].shape[0] if has_res else do
        layer_desc.append((di, do, has_res))
        add_full(p["pose"])
        add_full(p["ln_g"])
        add_full(p["ln_b"])
        if has_res:
            add_full(p["res_W"].astype(jnp.bfloat16))
        add_full(p["g_W"])
        add_full(p["g_b"])
        add_full(p["b_W"])
        add_full(p["b_b"])

    d_out = layers[-1]["pose"].shape[-1]
    rows = _JB * F
    scratch = [pltpu.VMEM((rows, do), jnp.float32)
               for (_di, do, _hr) in layer_desc]
    scratch += [pltpu.VMEM((B, do), jnp.float32)
                for (_di, do, _hr) in layer_desc]
    scratch += [pltpu.VMEM((B, do), jnp.float32)
                for (_di, do, _hr) in layer_desc]
    out = pl.pallas_call(
        lambda *refs: _encoder_body(tuple(layer_desc), J, F, B, *refs),
        grid=(nj, B),
        in_specs=specs,
        out_specs=pl.BlockSpec((1, _JB, F, d_out), lambda jc, b: (b, jc, 0, 0)),
        out_shape=jax.ShapeDtypeStruct((B, J, F, d_out), jnp.float32),
        scratch_shapes=scratch,
    )(*operands)
    return jnp.transpose(out, (0, 2, 1, 3))


# speedup vs baseline: 11.5219x; 1.0249x over previous
import jax
import jax.numpy as jnp
from jax.experimental import pallas as pl
from jax.experimental.pallas import tpu as pltpu

_JB = 27


def _encoder_body(layer_desc, n_joints, n_frames, n_batch, *refs):
    nl = len(layer_desc)
    theta_ref, cond_ref, in_w_ref, in_b_ref = refs[:4]
    out_ref = refs[-1 - 3 * nl]
    crefs = refs[-3 * nl:-2 * nl]
    grefs = refs[-2 * nl:-nl]
    brefs = refs[-nl:]
    lrefs = refs[4:-1 - 3 * nl]
    rows = _JB * n_frames
    jc = pl.program_id(0)
    b = pl.program_id(1)

    @pl.when(jnp.logical_and(jc == 0, b == 0))
    def _compute_gates():
        cond = cond_ref[...]
        i = 0
        for li, (_di, _do, has_res) in enumerate(layer_desc):
            i += 3
            if has_res:
                i += 1
            g_w = lrefs[i][...]; i += 1
            g_b = lrefs[i][...]; i += 1
            b_w = lrefs[i][...]; i += 1
            b_b = lrefs[i][...]; i += 1
            grefs[li][...] = jax.nn.sigmoid(
                jnp.dot(cond, g_w, preferred_element_type=jnp.float32) + g_b)
            brefs[li][...] = jnp.tanh(
                jnp.dot(cond, b_w, preferred_element_type=jnp.float32) + b_b)

    @pl.when(b == 0)
    def _compute_constants():
        j0 = jc * _JB
        row_j = j0 + jax.lax.broadcasted_iota(
            jnp.int32, (rows, n_joints), 0) // n_frames
        col_j = jax.lax.broadcasted_iota(jnp.int32, (rows, n_joints), 1)
        sel = jnp.where(row_j == col_j, 1.0, 0.0).astype(jnp.float32)
        i = 0
        for li, (_di, _do, has_res) in enumerate(layer_desc):
            pose = lrefs[i][...]; i += 1
            ln_g = lrefs[i][...]; i += 1
            ln_b = lrefs[i][...]; i += 1
            if has_res:
                i += 1
            i += 4
            mu = jnp.mean(pose, axis=-1, keepdims=True)
            var = jnp.mean(jnp.square(pose - mu), axis=-1, keepdims=True)
            normed = (pose - mu) * jax.lax.rsqrt(var + 1e-5) * ln_g + ln_b
            c = 0.5 * normed * (1.0 + jax.lax.erf(normed * (2.0 ** -0.5)))
            crefs[li][...] = jnp.dot(sel, c,
                                     preferred_element_type=jnp.float32)

    x3 = theta_ref[...]
    x = x3.reshape(rows, x3.shape[-1])
    h = jnp.dot(x, in_w_ref[...],
                preferred_element_type=jnp.float32) + in_b_ref[...]

    i = 0
    for li, (_di, _do, has_res) in enumerate(layer_desc):
        i += 3
        if has_res:
            res_w = lrefs[i][...]; i += 1
        i += 4
        g = grefs[li][pl.ds(b, 1), :]
        bta = brefs[li][pl.ds(b, 1), :]
        res = (jnp.dot(h.astype(jnp.bfloat16), res_w.astype(jnp.bfloat16),
                       preferred_element_type=jnp.float32)
               if has_res else h)
        h = (crefs[li][...] + res) * g + bta

    out_ref[...] = h.reshape(1, _JB, n_frames, h.shape[-1])


def kernel(theta, cond, in_W, in_b, layers):
    B, F, J, nf = theta.shape
    cond_dim = cond.shape[-1]
    nj = -(-J // _JB)

    theta_t = jnp.transpose(theta, (0, 2, 1, 3))

    layer_desc = []
    operands = [theta_t, cond, in_W, in_b.reshape(1, -1)]
    specs = [
        pl.BlockSpec((1, _JB, F, nf), lambda jc, b: (b, jc, 0, 0)),
        pl.BlockSpec((B, cond_dim), lambda jc, b: (0, 0)),
        pl.BlockSpec(in_W.shape, lambda jc, b: (0, 0)),
        pl.BlockSpec((1, in_b.shape[0]), lambda jc, b: (0, 0)),
    ]

    def add_full(arr):
        a2 = arr.reshape(1, -1) if arr.ndim == 1 else arr
        operands.append(a2)
        specs.append(pl.BlockSpec(a2.shape, lambda jc, b: (0,) * a2.ndim))

    for p in layers:
        do = p["pose"].shape[-1]
        has_res = p["res_W"] is not None
        di = p["res_W"].shape[0] if has_res else do
        layer_desc.append((di, do, has_res))
        add_full(p["pose"])
        add_full(p["ln_g"])
        add_full(p["ln_b"])
        if has_res:
            add_full(p["res_W"])
        add_full(p["g_W"])
        add_full(p["g_b"])
        add_full(p["b_W"])
        add_full(p["b_b"])

    d_out = layers[-1]["pose"].shape[-1]
    rows = _JB * F
    scratch = [pltpu.VMEM((rows, do), jnp.float32)
               for (_di, do, _hr) in layer_desc]
    scratch += [pltpu.VMEM((B, do), jnp.float32)
                for (_di, do, _hr) in layer_desc]
    scratch += [pltpu.VMEM((B, do), jnp.float32)
                for (_di, do, _hr) in layer_desc]
    out = pl.pallas_call(
        lambda *refs: _encoder_body(tuple(layer_desc), J, F, B, *refs),
        grid=(nj, B),
        in_specs=specs,
        out_specs=pl.BlockSpec((1, _JB, F, d_out), lambda jc, b: (b, jc, 0, 0)),
        out_shape=jax.ShapeDtypeStruct((B, J, F, d_out), jnp.float32),
        scratch_shapes=scratch,
    )(*operands)
    return jnp.transpose(out, (0, 2, 1, 3))
